# baseline (speedup 1.0000x reference)
import functools

import jax
import jax.numpy as jnp
import numpy as np
from jax import lax
from jax.experimental import pallas as pl
from jax.experimental.pallas import tpu as pltpu
from jax.experimental.pallas import tpu_sc as plsc

_NC = 2
_NS = 16
_CHUNK = 80


def _deinterleave_perm(H):
    perm = np.empty(H, dtype=np.int32)
    for j in range(H // 32):
        for i in range(16):
            perm[32 * j + i] = 32 * j + 2 * i
            perm[32 * j + 16 + i] = 32 * j + 2 * i + 1
    return perm


def _segment_sum_sc(hb_u, src, dst):
    N, Hu = hb_u.shape
    H = 2 * Hu
    E = src.shape[0]
    nw = _NC * _NS
    epw = E // nw
    nchunk = epw // _CHUNK
    assert epw * nw == E and nchunk * _CHUNK == epw
    wrows = 40
    nwc = N // wrows
    assert nwc * wrows == N

    mesh = plsc.VectorSubcoreMesh(core_axis_name="c", subcore_axis_name="s")

    R = 4
    assert (nchunk - 1) % R == 0

    F = 2

    scratch = (
        [pltpu.VMEM((_CHUNK,), jnp.int32) for _ in range(R)]
        + [pltpu.VMEM((_CHUNK,), jnp.int32) for _ in range(R)]
        + [pltpu.VMEM((_CHUNK, Hu), jnp.int32) for _ in range(R)]
        + [pltpu.VMEM((_CHUNK, H), jnp.float32) for _ in range(F)]
        + [
            pltpu.VMEM((wrows, H), jnp.float32),
            pltpu.VMEM_SHARED((N, H), jnp.float32),
        ]
        + [pltpu.SemaphoreType.DMA for _ in range(2 * R + F)]
    )

    @functools.partial(
        pl.kernel,
        out_type=jax.ShapeDtypeStruct((_NC, N, H), jnp.float32),
        mesh=mesh,
        scratch_types=scratch,
        compiler_params=pltpu.CompilerParams(use_tc_tiling_on_sc=False),
    )
    def seg(h_hbm, src_hbm, dst_hbm, out_hbm, *sc):
        sidx = sc[0:R]
        didx = sc[R:2 * R]
        rows = sc[2 * R:3 * R]
        rowf = sc[3 * R:3 * R + F]
        zbuf = sc[3 * R + F]
        agg = sc[3 * R + F + 1]
        sems = sc[3 * R + F + 2:]
        isem = sems[0:R]
        gsem = sems[R:2 * R]
        ssem = sems[2 * R:2 * R + F]

        cid = lax.axis_index("c")
        sid = lax.axis_index("s")
        wid = cid * _NS + sid
        ebase = wid * epw

        def fire_idx(c, b):
            off = ebase + c * _CHUNK
            pltpu.async_copy(src_hbm.at[pl.ds(off, _CHUNK)], sidx[b], isem[b])
            pltpu.async_copy(dst_hbm.at[pl.ds(off, _CHUNK)], didx[b], isem[b])

        def wait_idx(b):
            pltpu.make_async_copy(src_hbm.at[pl.ds(0, _CHUNK)], sidx[b],
                                  isem[b]).wait()
            pltpu.make_async_copy(dst_hbm.at[pl.ds(0, _CHUNK)], didx[b],
                                  isem[b]).wait()

        def fire_gather(b):
            pltpu.async_copy(h_hbm.at[sidx[b]], rows[b], gsem[b])

        def wait_gather(b):
            pltpu.make_async_copy(h_hbm.at[pl.ds(0, _CHUNK)], rows[b],
                                  gsem[b]).wait()

        def convert(b, f):
            mask_hi = jnp.full((16,), -65536, dtype=jnp.int32)

            @pl.loop(0, _CHUNK)
            def _row(r):
                for j in range(Hu // 16):
                    u = rows[b][r, pl.ds(j * 16, 16)]
                    lo = lax.bitcast_convert_type(u << 16, jnp.float32)
                    hi = lax.bitcast_convert_type(u & mask_hi, jnp.float32)
                    rowf[f][r, pl.ds(j * 32, 16)] = lo
                    rowf[f][r, pl.ds(j * 32 + 16, 16)] = hi

        def fire_scatter(b, f):
            pltpu.async_copy(rowf[f], agg.at[didx[b]], ssem[f], add=True)

        def wait_scatter(b, f):
            pltpu.make_async_copy(rowf[f], agg.at[didx[b]], ssem[f]).wait()

        fire_idx(0, 0)
        fire_idx(1, 1)
        wait_idx(0)
        fire_gather(0)

        zero16 = jnp.zeros((16,), jnp.float32)

        @pl.loop(0, wrows)
        def _zero_rows(i):
            for j in range(H // 16):
                zbuf[i, pl.ds(j * 16, 16)] = zero16

        for i in range((nwc + _NS - 1) // _NS):
            blk = sid + i * _NS

            @pl.when(blk < nwc)
            def _():
                pltpu.sync_copy(zbuf, agg.at[pl.ds(blk * wrows, wrows)])
        plsc.subcore_barrier()

        @pl.loop(0, nchunk - 1, step=R)
        def _edge_block(c):
            for k in range(R):
                j = c + k
                s1 = (k + 1) % R
                s2 = (k + 2) % R
                f = k % F

                @pl.when(j >= 2)
                def _():
                    wait_scatter(s2, f)

                @pl.when(j + 2 < nchunk)
                def _():
                    fire_idx(j + 2, s2)

                @pl.when(j + 1 < nchunk)
                def _():
                    wait_idx(s1)
                    fire_gather(s1)

                wait_gather(k)
                convert(k, f)
                fire_scatter(k, f)

        last = (nchunk - 1) % R
        wait_scatter((last + 2) % R, last % F)
        wait_gather(last)
        convert(last, last % F)
        fire_scatter(last, last % F)
        wait_scatter((last + 3) % R, (last + 1) % F)
        wait_scatter(last, last % F)

        plsc.subcore_barrier()
        for i in range((nwc + _NS - 1) // _NS):
            blk = sid + i * _NS

            @pl.when(blk < nwc)
            def _():
                pltpu.sync_copy(agg.at[pl.ds(blk * wrows, wrows)],
                                out_hbm.at[cid, pl.ds(blk * wrows, wrows)])

    return seg(hb_u, src, dst)


_BLK = 2000


def _input_proj(x, W, b, g, beta):
    N, D = x.shape
    H = W.shape[1]

    def body(x_ref, w_ref, b_ref, g_ref, bb_ref, o_ref, ob_ref):
        h = jnp.dot(x_ref[...], w_ref[...],
                    preferred_element_type=jnp.float32) + b_ref[...]
        h = jnp.maximum(h, 0.0)
        mu = jnp.mean(h, axis=1, keepdims=True)
        var = jnp.mean((h - mu) ** 2, axis=1, keepdims=True)
        out = (h - mu) * lax.rsqrt(var + 1e-5) * g_ref[...] + bb_ref[...]
        o_ref[...] = out
        ob_ref[...] = out.astype(jnp.bfloat16)

    return pl.pallas_call(
        body,
        grid=(N // _BLK,),
        in_specs=[
            pl.BlockSpec((_BLK, D), lambda i: (i, 0)),
            pl.BlockSpec((D, H), lambda i: (0, 0)),
            pl.BlockSpec((1, H), lambda i: (0, 0)),
            pl.BlockSpec((1, H), lambda i: (0, 0)),
            pl.BlockSpec((1, H), lambda i: (0, 0)),
        ],
        out_specs=[pl.BlockSpec((_BLK, H), lambda i: (i, 0)),
                   pl.BlockSpec((_BLK, H), lambda i: (i, 0))],
        out_shape=[jax.ShapeDtypeStruct((N, H), jnp.float32),
                   jax.ShapeDtypeStruct((N, H), jnp.bfloat16)],
    )(x, W, b.reshape(1, H), g.reshape(1, H), beta.reshape(1, H))


def _gin_mlp(h, agg2p, W1, b1, W2, b2, bn_g, bn_b):
    N, H = h.shape
    s = bn_g / np.sqrt(1.0 + 1e-5)
    W2f = W2 * s[None, :]
    b2f = b2 * s + bn_b
    W1p = W1[_deinterleave_perm(H), :]

    def body(h_ref, a0_ref, a1_ref, w1_ref, w1p_ref, b1_ref, w2_ref, b2_ref,
             o_ref, ob_ref):
        a = a0_ref[0] + a1_ref[0]
        z = (jnp.dot(h_ref[...], w1_ref[...],
                     preferred_element_type=jnp.float32)
             + jnp.dot(a, w1p_ref[...], preferred_element_type=jnp.float32)
             + b1_ref[...])
        z = jnp.maximum(z, 0.0)
        z = jnp.dot(z, w2_ref[...],
                    preferred_element_type=jnp.float32) + b2_ref[...]
        out = jnp.maximum(z, 0.0)
        o_ref[...] = out
        ob_ref[...] = out.astype(jnp.bfloat16)

    return pl.pallas_call(
        body,
        grid=(N // _BLK,),
        in_specs=[
            pl.BlockSpec((_BLK, H), lambda i: (i, 0)),
            pl.BlockSpec((1, _BLK, H), lambda i: (0, i, 0)),
            pl.BlockSpec((1, _BLK, H), lambda i: (1, i, 0)),
            pl.BlockSpec((H, H), lambda i: (0, 0)),
            pl.BlockSpec((H, H), lambda i: (0, 0)),
            pl.BlockSpec((1, H), lambda i: (0, 0)),
            pl.BlockSpec((H, H), lambda i: (0, 0)),
            pl.BlockSpec((1, H), lambda i: (0, 0)),
        ],
        out_specs=[pl.BlockSpec((_BLK, H), lambda i: (i, 0)),
                   pl.BlockSpec((_BLK, H), lambda i: (i, 0))],
        out_shape=[jax.ShapeDtypeStruct((N, H), jnp.float32),
                   jax.ShapeDtypeStruct((N, H), jnp.bfloat16)],
    )(h, agg2p, agg2p, W1, W1p, b1.reshape(1, H), W2f, b2f.reshape(1, H))


def _pool(h, Wp, bp):
    N, H = h.shape

    def body(h_ref, wp_ref, bp_ref, o_ref):
        hm = jnp.mean(h_ref[...], axis=0, keepdims=True)
        hx = jnp.max(h_ref[...], axis=0, keepdims=True)
        hc = jnp.concatenate([hm, hx], axis=1)
        o = jnp.dot(hc, wp_ref[...],
                    preferred_element_type=jnp.float32) + bp_ref[...]
        o_ref[...] = jnp.maximum(o, 0.0)

    return pl.pallas_call(
        body,
        out_shape=jax.ShapeDtypeStruct((1, H), jnp.float32),
    )(h, Wp, bp.reshape(1, H))


def kernel(x, edge_index, params):
    src = edge_index[0]
    dst = edge_index[1]
    h, hb = _input_proj(x, params["W_in"], params["b_in"],
                        params["ln_g"], params["ln_b"])
    for lp in params["layers"]:
        N, H = hb.shape
        hb_u = jax.lax.bitcast_convert_type(
            hb.reshape(N, H // 2, 2), jnp.int32)
        agg2p = _segment_sum_sc(hb_u, src, dst)
        h, hb = _gin_mlp(h, agg2p, lp["W1"], lp["b1"],
                         lp["W2"], lp["b2"], lp["bn_g"], lp["bn_b"])
    return _pool(h, params["W_pool"], params["b_pool"])


# speedup vs baseline: 2.1371x; 2.1371x over previous
import functools

import jax
import jax.numpy as jnp
import numpy as np
from jax import lax
from jax.experimental import pallas as pl
from jax.experimental.pallas import tpu as pltpu
from jax.experimental.pallas import tpu_sc as plsc

_NC = 2
_NS = 16
_CHUNK = 80


def _segment_sum_sc(h, src, dst):
    N, H = h.shape
    E = src.shape[0]
    nw = _NC * _NS
    epw = E // nw
    nchunk = epw // _CHUNK
    assert epw * nw == E and nchunk * _CHUNK == epw
    wrows = 40
    nwc = N // wrows
    assert nwc * wrows == N

    mesh = plsc.VectorSubcoreMesh(core_axis_name="c", subcore_axis_name="s")

    R = 4
    assert (nchunk - 1) % R == 0

    scratch = (
        [pltpu.VMEM((_CHUNK,), jnp.int32) for _ in range(R)]
        + [pltpu.VMEM((_CHUNK,), jnp.int32) for _ in range(R)]
        + [pltpu.VMEM((_CHUNK, H), jnp.float32) for _ in range(R)]
        + [
            pltpu.VMEM((wrows, H), jnp.float32),
            pltpu.VMEM_SHARED((N, H), jnp.float32),
        ]
        + [pltpu.SemaphoreType.DMA for _ in range(3 * R)]
    )

    @functools.partial(
        pl.kernel,
        out_type=jax.ShapeDtypeStruct((_NC, N, H), jnp.float32),
        mesh=mesh,
        scratch_types=scratch,
    )
    def seg(h_hbm, src_hbm, dst_hbm, out_hbm, *sc):
        sidx = sc[0:R]
        didx = sc[R:2 * R]
        rows = sc[2 * R:3 * R]
        zbuf = sc[3 * R]
        agg = sc[3 * R + 1]
        sems = sc[3 * R + 2:]
        isem = sems[0:R]
        gsem = sems[R:2 * R]
        ssem = sems[2 * R:3 * R]

        cid = lax.axis_index("c")
        sid = lax.axis_index("s")
        wid = cid * _NS + sid
        ebase = wid * epw

        def fire_idx(c, b):
            off = ebase + c * _CHUNK
            pltpu.async_copy(src_hbm.at[pl.ds(off, _CHUNK)], sidx[b], isem[b])
            pltpu.async_copy(dst_hbm.at[pl.ds(off, _CHUNK)], didx[b], isem[b])

        def wait_idx(b):
            pltpu.make_async_copy(src_hbm.at[pl.ds(0, _CHUNK)], sidx[b],
                                  isem[b]).wait()
            pltpu.make_async_copy(dst_hbm.at[pl.ds(0, _CHUNK)], didx[b],
                                  isem[b]).wait()

        def fire_gather(b):
            pltpu.async_copy(h_hbm.at[sidx[b]], rows[b], gsem[b])

        def wait_gather(b):
            pltpu.make_async_copy(h_hbm.at[pl.ds(0, _CHUNK)], rows[b],
                                  gsem[b]).wait()

        def fire_scatter(b):
            pltpu.async_copy(rows[b], agg.at[didx[b]], ssem[b], add=True)

        def wait_scatter(b):
            pltpu.make_async_copy(rows[b], agg.at[didx[b]], ssem[b]).wait()

        fire_idx(0, 0)
        fire_idx(1, 1)
        wait_idx(0)
        fire_gather(0)

        zero16 = jnp.zeros((16,), jnp.float32)

        @pl.loop(0, wrows)
        def _zero_rows(i):
            for j in range(H // 16):
                zbuf[i, pl.ds(j * 16, 16)] = zero16

        for i in range((nwc + _NS - 1) // _NS):
            blk = sid + i * _NS

            @pl.when(blk < nwc)
            def _():
                pltpu.sync_copy(zbuf, agg.at[pl.ds(blk * wrows, wrows)])
        plsc.subcore_barrier()

        @pl.loop(0, nchunk - 1, step=R)
        def _edge_block(c):
            for k in range(R):
                j = c + k
                s1 = (k + 1) % R
                s2 = (k + 2) % R

                @pl.when(j >= 2)
                def _():
                    wait_scatter(s2)

                @pl.when(j + 2 < nchunk)
                def _():
                    fire_idx(j + 2, s2)

                @pl.when(j + 1 < nchunk)
                def _():
                    wait_idx(s1)
                    fire_gather(s1)

                wait_gather(k)
                fire_scatter(k)

        last = (nchunk - 1) % R
        wait_gather(last)
        fire_scatter(last)
        wait_scatter((last + 2) % R)
        wait_scatter((last + 3) % R)
        wait_scatter(last)

        plsc.subcore_barrier()
        for i in range((nwc + _NS - 1) // _NS):
            blk = sid + i * _NS

            @pl.when(blk < nwc)
            def _():
                pltpu.sync_copy(agg.at[pl.ds(blk * wrows, wrows)],
                                out_hbm.at[cid, pl.ds(blk * wrows, wrows)])

    return seg(h, src, dst)


_BLK = 2000


def _input_proj(x, W, b, g, beta):
    N, D = x.shape
    H = W.shape[1]

    def body(x_ref, w_ref, b_ref, g_ref, bb_ref, o_ref):
        h = jnp.dot(x_ref[...], w_ref[...],
                    preferred_element_type=jnp.float32) + b_ref[...]
        h = jnp.maximum(h, 0.0)
        mu = jnp.mean(h, axis=1, keepdims=True)
        var = jnp.mean((h - mu) ** 2, axis=1, keepdims=True)
        o_ref[...] = (h - mu) * lax.rsqrt(var + 1e-5) * g_ref[...] + bb_ref[...]

    return pl.pallas_call(
        body,
        grid=(N // _BLK,),
        in_specs=[
            pl.BlockSpec((_BLK, D), lambda i: (i, 0)),
            pl.BlockSpec((D, H), lambda i: (0, 0)),
            pl.BlockSpec((1, H), lambda i: (0, 0)),
            pl.BlockSpec((1, H), lambda i: (0, 0)),
            pl.BlockSpec((1, H), lambda i: (0, 0)),
        ],
        out_specs=pl.BlockSpec((_BLK, H), lambda i: (i, 0)),
        out_shape=jax.ShapeDtypeStruct((N, H), jnp.float32),
    )(x, W, b.reshape(1, H), g.reshape(1, H), beta.reshape(1, H))


def _gin_mlp(h, agg2, W1, b1, W2, b2, bn_g, bn_b):
    N, H = h.shape
    s = bn_g / np.sqrt(1.0 + 1e-5)
    W2f = W2 * s[None, :]
    b2f = b2 * s + bn_b

    def body(h_ref, a0_ref, a1_ref, w1_ref, b1_ref, w2_ref, b2_ref, o_ref):
        z = h_ref[...] + a0_ref[0] + a1_ref[0]
        z = jnp.dot(z, w1_ref[...],
                    preferred_element_type=jnp.float32) + b1_ref[...]
        z = jnp.maximum(z, 0.0)
        z = jnp.dot(z, w2_ref[...],
                    preferred_element_type=jnp.float32) + b2_ref[...]
        o_ref[...] = jnp.maximum(z, 0.0)

    return pl.pallas_call(
        body,
        grid=(N // _BLK,),
        in_specs=[
            pl.BlockSpec((_BLK, H), lambda i: (i, 0)),
            pl.BlockSpec((1, _BLK, H), lambda i: (0, i, 0)),
            pl.BlockSpec((1, _BLK, H), lambda i: (1, i, 0)),
            pl.BlockSpec((H, H), lambda i: (0, 0)),
            pl.BlockSpec((1, H), lambda i: (0, 0)),
            pl.BlockSpec((H, H), lambda i: (0, 0)),
            pl.BlockSpec((1, H), lambda i: (0, 0)),
        ],
        out_specs=pl.BlockSpec((_BLK, H), lambda i: (i, 0)),
        out_shape=jax.ShapeDtypeStruct((N, H), jnp.float32),
    )(h, agg2, agg2, W1, b1.reshape(1, H), W2f, b2f.reshape(1, H))


def _pool(h, Wp, bp):
    N, H = h.shape

    def body(h_ref, wp_ref, bp_ref, o_ref):
        hm = jnp.mean(h_ref[...], axis=0, keepdims=True)
        hx = jnp.max(h_ref[...], axis=0, keepdims=True)
        hc = jnp.concatenate([hm, hx], axis=1)
        o = jnp.dot(hc, wp_ref[...],
                    preferred_element_type=jnp.float32) + bp_ref[...]
        o_ref[...] = jnp.maximum(o, 0.0)

    return pl.pallas_call(
        body,
        out_shape=jax.ShapeDtypeStruct((1, H), jnp.float32),
    )(h, Wp, bp.reshape(1, H))


def kernel(x, edge_index, params):
    src = edge_index[0]
    dst = edge_index[1]
    h = _input_proj(x, params["W_in"], params["b_in"],
                    params["ln_g"], params["ln_b"])
    for lp in params["layers"]:
        agg2 = _segment_sum_sc(h, src, dst)
        h = _gin_mlp(h, agg2, lp["W1"], lp["b1"],
                     lp["W2"], lp["b2"], lp["bn_g"], lp["bn_b"])
    return _pool(h, params["W_pool"], params["b_pool"])


# speedup vs baseline: 2.2395x; 1.0479x over previous
import functools

import jax
import jax.numpy as jnp
import numpy as np
from jax import lax
from jax.experimental import pallas as pl
from jax.experimental.pallas import tpu as pltpu
from jax.experimental.pallas import tpu_sc as plsc

_NC = 2
_NS = 16
_CHUNK = 80


def _segment_sum_sc(h, src, dst):
    N, H = h.shape
    E = src.shape[0]
    nw = _NC * _NS
    epw = E // nw
    nchunk = epw // _CHUNK
    assert epw * nw == E and nchunk * _CHUNK == epw
    wrows = 40
    nwc = N // wrows
    assert nwc * wrows == N

    mesh = plsc.VectorSubcoreMesh(core_axis_name="c", subcore_axis_name="s")

    R = 4
    assert (nchunk - 1) % R == 0

    scratch = (
        [pltpu.VMEM((_CHUNK,), jnp.int32) for _ in range(R)]
        + [pltpu.VMEM((_CHUNK,), jnp.int32) for _ in range(R)]
        + [pltpu.VMEM((_CHUNK, H), jnp.float32) for _ in range(R)]
        + [
            pltpu.VMEM((wrows, H), jnp.float32),
            pltpu.VMEM_SHARED((N, H), jnp.float32),
        ]
        + [pltpu.SemaphoreType.DMA for _ in range(3 * R + 1)]
    )

    @functools.partial(
        pl.kernel,
        out_type=jax.ShapeDtypeStruct((_NC, N, H), jnp.float32),
        mesh=mesh,
        scratch_types=scratch,
    )
    def seg(h_hbm, src_hbm, dst_hbm, out_hbm, *sc):
        sidx = sc[0:R]
        didx = sc[R:2 * R]
        rows = sc[2 * R:3 * R]
        zbuf = sc[3 * R]
        agg = sc[3 * R + 1]
        sems = sc[3 * R + 2:]
        isem = sems[0:R]
        gsem = sems[R:2 * R]
        ssem = sems[2 * R:3 * R]
        zsem = sems[3 * R]

        cid = lax.axis_index("c")
        sid = lax.axis_index("s")
        wid = cid * _NS + sid
        ebase = wid * epw

        def fire_idx(c, b):
            off = ebase + c * _CHUNK
            pltpu.async_copy(src_hbm.at[pl.ds(off, _CHUNK)], sidx[b], isem[b])
            pltpu.async_copy(dst_hbm.at[pl.ds(off, _CHUNK)], didx[b], isem[b])

        def wait_idx(b):
            pltpu.make_async_copy(src_hbm.at[pl.ds(0, _CHUNK)], sidx[b],
                                  isem[b]).wait()
            pltpu.make_async_copy(dst_hbm.at[pl.ds(0, _CHUNK)], didx[b],
                                  isem[b]).wait()

        def fire_gather(b):
            pltpu.async_copy(h_hbm.at[sidx[b]], rows[b], gsem[b])

        def wait_gather(b):
            pltpu.make_async_copy(h_hbm.at[pl.ds(0, _CHUNK)], rows[b],
                                  gsem[b]).wait()

        def fire_scatter(b):
            pltpu.async_copy(rows[b], agg.at[didx[b]], ssem[b], add=True)

        def wait_scatter(b):
            pltpu.make_async_copy(rows[b], agg.at[didx[b]], ssem[b]).wait()

        fire_idx(0, 0)
        fire_idx(1, 1)
        wait_idx(0)
        fire_gather(0)

        zero16 = jnp.zeros((16,), jnp.float32)

        @pl.loop(0, wrows)
        def _zero_rows(i):
            for j in range(H // 16):
                zbuf[i, pl.ds(j * 16, 16)] = zero16

        for i in range((nwc + _NS - 1) // _NS):
            blk = sid + i * _NS

            @pl.when(blk < nwc)
            def _():
                pltpu.async_copy(zbuf, agg.at[pl.ds(blk * wrows, wrows)],
                                 zsem)
        for i in range((nwc + _NS - 1) // _NS):
            blk = sid + i * _NS

            @pl.when(blk < nwc)
            def _():
                pltpu.make_async_copy(
                    zbuf, agg.at[pl.ds(blk * wrows, wrows)], zsem).wait()
        plsc.subcore_barrier()

        @pl.loop(0, nchunk - 1, step=R)
        def _edge_block(c):
            for k in range(R):
                j = c + k
                s1 = (k + 1) % R
                s2 = (k + 2) % R

                @pl.when(j >= 2)
                def _():
                    wait_scatter(s2)

                @pl.when(j + 2 < nchunk)
                def _():
                    fire_idx(j + 2, s2)

                @pl.when(j + 1 < nchunk)
                def _():
                    wait_idx(s1)
                    fire_gather(s1)

                wait_gather(k)
                fire_scatter(k)

        last = (nchunk - 1) % R
        wait_gather(last)
        fire_scatter(last)
        wait_scatter((last + 2) % R)
        wait_scatter((last + 3) % R)
        wait_scatter(last)

        plsc.subcore_barrier()
        for i in range((nwc + _NS - 1) // _NS):
            blk = sid + i * _NS

            @pl.when(blk < nwc)
            def _():
                pltpu.async_copy(agg.at[pl.ds(blk * wrows, wrows)],
                                 out_hbm.at[cid, pl.ds(blk * wrows, wrows)],
                                 zsem)
        for i in range((nwc + _NS - 1) // _NS):
            blk = sid + i * _NS

            @pl.when(blk < nwc)
            def _():
                pltpu.make_async_copy(
                    agg.at[pl.ds(blk * wrows, wrows)],
                    out_hbm.at[cid, pl.ds(blk * wrows, wrows)], zsem).wait()

    return seg(h, src, dst)


_BLK = 2000


def _input_proj(x, W, b, g, beta):
    N, D = x.shape
    H = W.shape[1]

    def body(x_ref, w_ref, b_ref, g_ref, bb_ref, o_ref):
        h = jnp.dot(x_ref[...], w_ref[...],
                    preferred_element_type=jnp.float32) + b_ref[...]
        h = jnp.maximum(h, 0.0)
        mu = jnp.mean(h, axis=1, keepdims=True)
        var = jnp.mean((h - mu) ** 2, axis=1, keepdims=True)
        o_ref[...] = (h - mu) * lax.rsqrt(var + 1e-5) * g_ref[...] + bb_ref[...]

    return pl.pallas_call(
        body,
        grid=(N // _BLK,),
        in_specs=[
            pl.BlockSpec((_BLK, D), lambda i: (i, 0)),
            pl.BlockSpec((D, H), lambda i: (0, 0)),
            pl.BlockSpec((1, H), lambda i: (0, 0)),
            pl.BlockSpec((1, H), lambda i: (0, 0)),
            pl.BlockSpec((1, H), lambda i: (0, 0)),
        ],
        out_specs=pl.BlockSpec((_BLK, H), lambda i: (i, 0)),
        out_shape=jax.ShapeDtypeStruct((N, H), jnp.float32),
    )(x, W, b.reshape(1, H), g.reshape(1, H), beta.reshape(1, H))


def _gin_mlp(h, agg2, W1, b1, W2, b2, bn_g, bn_b, pool=None):
    N, H = h.shape
    s = bn_g / np.sqrt(1.0 + 1e-5)
    W2f = W2 * s[None, :]
    b2f = b2 * s + bn_b
    grid = N // _BLK

    def mlp_block(h_ref, a0_ref, a1_ref, w1_ref, b1_ref, w2_ref, b2_ref):
        z = h_ref[...] + a0_ref[0] + a1_ref[0]
        z = jnp.dot(z, w1_ref[...],
                    preferred_element_type=jnp.float32) + b1_ref[...]
        z = jnp.maximum(z, 0.0)
        z = jnp.dot(z, w2_ref[...],
                    preferred_element_type=jnp.float32) + b2_ref[...]
        return jnp.maximum(z, 0.0)

    in_specs = [
        pl.BlockSpec((_BLK, H), lambda i: (i, 0)),
        pl.BlockSpec((1, _BLK, H), lambda i: (0, i, 0)),
        pl.BlockSpec((1, _BLK, H), lambda i: (1, i, 0)),
        pl.BlockSpec((H, H), lambda i: (0, 0)),
        pl.BlockSpec((1, H), lambda i: (0, 0)),
        pl.BlockSpec((H, H), lambda i: (0, 0)),
        pl.BlockSpec((1, H), lambda i: (0, 0)),
    ]
    args = [h, agg2, agg2, W1, b1.reshape(1, H), W2f, b2f.reshape(1, H)]

    if pool is None:
        def body(*refs):
            refs[-1][...] = mlp_block(*refs[:-1])

        return pl.pallas_call(
            body,
            grid=(grid,),
            in_specs=in_specs,
            out_specs=pl.BlockSpec((_BLK, H), lambda i: (i, 0)),
            out_shape=jax.ShapeDtypeStruct((N, H), jnp.float32),
        )(*args)

    Wp, bp = pool

    def body_pool(h_ref, a0_ref, a1_ref, w1_ref, b1_ref, w2_ref, b2_ref,
                  wp_ref, bp_ref, o_ref, p_ref, acc_s, acc_m):
        i = pl.program_id(0)
        out = mlp_block(h_ref, a0_ref, a1_ref, w1_ref, b1_ref, w2_ref, b2_ref)
        o_ref[...] = out
        bsum = jnp.sum(out, axis=0, keepdims=True)
        bmax = jnp.max(out, axis=0, keepdims=True)

        @pl.when(i == 0)
        def _():
            acc_s[...] = bsum
            acc_m[...] = bmax

        @pl.when(i > 0)
        def _():
            acc_s[...] = acc_s[...] + bsum
            acc_m[...] = jnp.maximum(acc_m[...], bmax)

        @pl.when(i == grid - 1)
        def _():
            hc = jnp.concatenate([acc_s[...] * (1.0 / N), acc_m[...]], axis=1)
            p_ref[...] = jnp.maximum(
                jnp.dot(hc, wp_ref[...],
                        preferred_element_type=jnp.float32) + bp_ref[...],
                0.0)

    return pl.pallas_call(
        body_pool,
        grid=(grid,),
        in_specs=in_specs + [
            pl.BlockSpec((2 * H, H), lambda i: (0, 0)),
            pl.BlockSpec((1, H), lambda i: (0, 0)),
        ],
        out_specs=[pl.BlockSpec((_BLK, H), lambda i: (i, 0)),
                   pl.BlockSpec((1, H), lambda i: (0, 0))],
        out_shape=[jax.ShapeDtypeStruct((N, H), jnp.float32),
                   jax.ShapeDtypeStruct((1, H), jnp.float32)],
        scratch_shapes=[pltpu.VMEM((1, H), jnp.float32),
                        pltpu.VMEM((1, H), jnp.float32)],
    )(*args, Wp, bp.reshape(1, H))


def kernel(x, edge_index, params):
    src = edge_index[0]
    dst = edge_index[1]
    h = _input_proj(x, params["W_in"], params["b_in"],
                    params["ln_g"], params["ln_b"])
    layers = params["layers"]
    for lp in layers[:-1]:
        agg2 = _segment_sum_sc(h, src, dst)
        h = _gin_mlp(h, agg2, lp["W1"], lp["b1"],
                     lp["W2"], lp["b2"], lp["bn_g"], lp["bn_b"])
    lp = layers[-1]
    agg2 = _segment_sum_sc(h, src, dst)
    _, pooled = _gin_mlp(h, agg2, lp["W1"], lp["b1"],
                         lp["W2"], lp["b2"], lp["bn_g"], lp["bn_b"],
                         pool=(params["W_pool"], params["b_pool"]))
    return pooled
